# Initial kernel scaffold; baseline (speedup 1.0000x reference)
import jax
import jax.numpy as jnp
from jax.experimental import pallas as pl


def kernel(hidden_states, weight):
    raise NotImplementedError("write your pallas kernel here")


# speedup vs baseline: 1.3096x; 1.3096x over previous
import jax
import jax.numpy as jnp
from jax.experimental import pallas as pl
from jax.experimental.pallas import tpu as pltpu

_EPS = 1e-06
_BLOCK_ROWS = 256


def _fused_body(h_ref, w_ref, norm_ref, x_ref):
    z = jnp.maximum(h_ref[...], 0.0)
    x = z + z
    var = jnp.mean(x * x, axis=-1, keepdims=True)
    inv = jax.lax.rsqrt(var + _EPS)
    x_ref[...] = x
    norm_ref[...] = (x * inv) * w_ref[...]


def kernel(hidden_states, weight):
    tokens, hidden = hidden_states.shape
    w2d = weight.reshape(1, hidden)
    grid = (tokens // _BLOCK_ROWS,)
    norm, x = pl.pallas_call(
        _fused_body,
        out_shape=(
            jax.ShapeDtypeStruct((tokens, hidden), hidden_states.dtype),
            jax.ShapeDtypeStruct((tokens, hidden), hidden_states.dtype),
        ),
        grid=grid,
        in_specs=[
            pl.BlockSpec((_BLOCK_ROWS, hidden), lambda i: (i, 0)),
            pl.BlockSpec((1, hidden), lambda i: (0, 0)),
        ],
        out_specs=(
            pl.BlockSpec((_BLOCK_ROWS, hidden), lambda i: (i, 0)),
            pl.BlockSpec((_BLOCK_ROWS, hidden), lambda i: (i, 0)),
        ),
        compiler_params=pltpu.CompilerParams(
            dimension_semantics=("parallel",),
        ),
        name="fused_relu_add_rmsnorm",
    )(hidden_states, w2d)
    return (norm, x)


# speedup vs baseline: 1.3456x; 1.0275x over previous
import jax
import jax.numpy as jnp
from jax.experimental import pallas as pl
from jax.experimental.pallas import tpu as pltpu

_EPS = 1e-06
_BLOCK_ROWS = 512


def _fused_body(h_ref, w_ref, norm_ref, x_ref):
    z = jnp.maximum(h_ref[...], 0.0)
    x = z + z
    var = jnp.mean(x * x, axis=-1, keepdims=True)
    inv = jax.lax.rsqrt(var + _EPS)
    x_ref[...] = x
    norm_ref[...] = (x * inv) * w_ref[...]


def kernel(hidden_states, weight):
    tokens, hidden = hidden_states.shape
    w2d = weight.reshape(1, hidden)
    grid = (tokens // _BLOCK_ROWS,)
    norm, x = pl.pallas_call(
        _fused_body,
        out_shape=(
            jax.ShapeDtypeStruct((tokens, hidden), hidden_states.dtype),
            jax.ShapeDtypeStruct((tokens, hidden), hidden_states.dtype),
        ),
        grid=grid,
        in_specs=[
            pl.BlockSpec((_BLOCK_ROWS, hidden), lambda i: (i, 0)),
            pl.BlockSpec((1, hidden), lambda i: (0, 0)),
        ],
        out_specs=(
            pl.BlockSpec((_BLOCK_ROWS, hidden), lambda i: (i, 0)),
            pl.BlockSpec((_BLOCK_ROWS, hidden), lambda i: (i, 0)),
        ),
        compiler_params=pltpu.CompilerParams(
            dimension_semantics=("parallel",),
        ),
        name="fused_relu_add_rmsnorm",
    )(hidden_states, w2d)
    return (norm, x)
